# baseline (speedup 1.0000x reference)
import jax
import jax.numpy as jnp
from jax import lax
from jax.experimental import pallas as pl
from jax.experimental.pallas import tpu as pltpu
from jax.experimental.pallas import tpu_sc as plsc

VOCAB = 10000
DIM = 512
SEQ = 2048
BATCH = 64
N = SEQ * BATCH
N_POS = 13
N_FPOS = 5
N_COMBO = N_POS * N_FPOS

NC = 2
NS = 16
NW = NC * NS
PER_W = N // NW
CHUNK = 32
N_CHUNKS = PER_W // CHUNK
NB = 4
LOOKAHEAD = 2
LANES = 16
DV = DIM // LANES


def _embed_kernel(tok_hbm, pos_hbm, fpos_hbm, emb_hbm, post_hbm, fpost_hbm,
                  out_hbm, idx_t, idx_c, idx_f, rows, pos_v, fpos_v, combo_v,
                  sem_g0, sem_g1, sem_g2, sem_g3,
                  sem_o0, sem_o1, sem_o2, sem_o3, sem_i):
    sem_g = [sem_g0, sem_g1, sem_g2, sem_g3]
    sem_o = [sem_o0, sem_o1, sem_o2, sem_o3]
    wid = lax.axis_index("s") * NC + lax.axis_index("c")
    w_base = wid * PER_W

    h1 = pltpu.async_copy(tok_hbm.at[pl.ds(w_base, PER_W)], idx_t, sem_i)
    h2 = pltpu.async_copy(pos_hbm.at[pl.ds(w_base, PER_W)],
                          idx_c.at[pl.ds(0, PER_W)], sem_i)
    h3 = pltpu.async_copy(fpos_hbm.at[pl.ds(w_base, PER_W)], idx_f, sem_i)

    pltpu.sync_copy(post_hbm, pos_v)
    pltpu.sync_copy(fpost_hbm, fpos_v)

    @plsc.parallel_loop(0, N_COMBO, 1, unroll=2)
    def build_combo(c):
        i = c // N_FPOS
        j = c - i * N_FPOS
        for k in range(DV):
            sl = pl.ds(k * LANES, LANES)
            combo_v[c, sl] = pos_v[i, sl] + fpos_v[j, sl]

    h1.wait()
    h2.wait()
    h3.wait()

    @plsc.parallel_loop(0, PER_W // LANES, 1, unroll=4)
    def fuse_idx(k):
        sl = pl.ds(k * LANES, LANES)
        idx_c[sl] = idx_c[sl] * N_FPOS + idx_f[sl]

    def issue_gather(ci, b):
        pltpu.async_copy(emb_hbm.at[idx_t.at[pl.ds(ci * CHUNK, CHUNK)]],
                         rows.at[b], sem_g[b])

    def wait_gather(b):
        pltpu.make_async_copy(emb_hbm.at[pl.ds(0, CHUNK)], rows.at[b],
                              sem_g[b]).wait()

    def issue_scatter(ci, b):
        base = w_base + ci * CHUNK
        pltpu.async_copy(rows.at[b], out_hbm.at[pl.ds(base, CHUNK)], sem_o[b])

    def wait_scatter(b):
        pltpu.make_async_copy(rows.at[b], out_hbm.at[pl.ds(0, CHUNK)],
                              sem_o[b]).wait()

    def add_pass(ci, b):
        @plsc.parallel_loop(0, CHUNK, 1, unroll=4)
        def add_row(r):
            c = idx_c[pl.ds(ci * CHUNK + r, LANES)][0]
            for k in range(DV):
                sl = pl.ds(k * LANES, LANES)
                plsc.addupdate(rows.at[b, r, sl], combo_v[c, sl])

    for j in range(LOOKAHEAD):
        issue_gather(j, j)

    def outer(go, carry):
        for b in range(NB):
            i = go * NB + b

            @pl.when(i + LOOKAHEAD < N_CHUNKS)
            def _next():
                nb = (b + LOOKAHEAD) % NB

                @pl.when(i + LOOKAHEAD >= NB)
                def _drain():
                    wait_scatter(nb)

                issue_gather(i + LOOKAHEAD, nb)

            wait_gather(b)
            add_pass(i, b)
            issue_scatter(i, b)
        return carry

    lax.fori_loop(0, N_CHUNKS // NB, outer, 0)
    for b in range(NB):
        wait_scatter(b)


@jax.jit
def _run(tok, pos, fpos, emb_table, pos_table, fpos_table):
    mesh = plsc.VectorSubcoreMesh(core_axis_name="c", subcore_axis_name="s")
    call = pl.kernel(
        _embed_kernel,
        mesh=mesh,
        out_type=jax.ShapeDtypeStruct((N, DIM), jnp.float32),
        scratch_types=[
            pltpu.VMEM((PER_W,), jnp.int32),
            pltpu.VMEM((PER_W + LANES,), jnp.int32),
            pltpu.VMEM((PER_W,), jnp.int32),
            pltpu.VMEM((NB, CHUNK, DIM), jnp.float32),
            pltpu.VMEM((N_POS, DIM), jnp.float32),
            pltpu.VMEM((N_FPOS, DIM), jnp.float32),
            pltpu.VMEM((N_COMBO, DIM), jnp.float32),
            pltpu.SemaphoreType.DMA,
            pltpu.SemaphoreType.DMA,
            pltpu.SemaphoreType.DMA,
            pltpu.SemaphoreType.DMA,
            pltpu.SemaphoreType.DMA,
            pltpu.SemaphoreType.DMA,
            pltpu.SemaphoreType.DMA,
            pltpu.SemaphoreType.DMA,
            pltpu.SemaphoreType.DMA,
        ],
    )
    out = call(tok, pos, fpos, emb_table, pos_table, fpos_table)
    return out.reshape(SEQ, BATCH, DIM)


def kernel(batch_datasets, batch_positionals, batch_float_positionals,
           emb_table, pos_table, fpos_table):
    tok = batch_datasets.reshape(N).astype(jnp.int32)
    pos = batch_positionals.reshape(N).astype(jnp.int32)
    fpos = batch_float_positionals.reshape(N).astype(jnp.int32)
    return _run(tok, pos, fpos, emb_table, pos_table, fpos_table)


# speedup vs baseline: 1.4273x; 1.4273x over previous
import jax
import jax.numpy as jnp
from jax import lax
from jax.experimental import pallas as pl
from jax.experimental.pallas import tpu as pltpu
from jax.experimental.pallas import tpu_sc as plsc

VOCAB = 10000
DIM = 512
SEQ = 2048
BATCH = 64
N = SEQ * BATCH
N_POS = 13
N_FPOS = 5
N_COMBO = N_POS * N_FPOS

NC = 2
NS = 16
NW = NC * NS
PER_W = N // NW
CHUNK = 32
N_CHUNKS = PER_W // CHUNK
NB = 4
LOOKAHEAD = 2
LANES = 16
DV = DIM // LANES


def _embed_kernel(tok_hbm, pos_hbm, fpos_hbm, emb_hbm, post_hbm, fpost_hbm,
                  out_hbm, idx_t, idx_c, idx_f, rows, pos_v, fpos_v, combo_v,
                  sem_g0, sem_g1, sem_g2, sem_g3,
                  sem_o0, sem_o1, sem_o2, sem_o3, sem_i):
    sem_g = [sem_g0, sem_g1, sem_g2, sem_g3]
    sem_o = [sem_o0, sem_o1, sem_o2, sem_o3]
    wid = lax.axis_index("s") * NC + lax.axis_index("c")
    w_base = wid * PER_W

    h1 = pltpu.async_copy(tok_hbm.at[pl.ds(w_base, PER_W)], idx_t, sem_i)
    h2 = pltpu.async_copy(pos_hbm.at[pl.ds(w_base, PER_W)],
                          idx_c.at[pl.ds(0, PER_W)], sem_i)
    h3 = pltpu.async_copy(fpos_hbm.at[pl.ds(w_base, PER_W)], idx_f, sem_i)

    pltpu.sync_copy(post_hbm, pos_v)
    pltpu.sync_copy(fpost_hbm, fpos_v)

    @plsc.parallel_loop(0, N_COMBO, 1, unroll=2)
    def build_combo(c):
        i = c // N_FPOS
        j = c - i * N_FPOS
        for k in range(DV):
            sl = pl.ds(k * LANES, LANES)
            combo_v[c, sl] = pos_v[i, sl] + fpos_v[j, sl]

    h1.wait()
    h2.wait()
    h3.wait()

    @plsc.parallel_loop(0, PER_W // LANES, 1, unroll=4)
    def fuse_idx(k):
        sl = pl.ds(k * LANES, LANES)
        idx_c[sl] = idx_c[sl] * N_FPOS + idx_f[sl]

    def issue_gather(ci, b):
        pltpu.async_copy(emb_hbm.at[idx_t.at[pl.ds(ci * CHUNK, CHUNK)]],
                         rows.at[b], sem_g[b])

    def wait_gather(b):
        pltpu.make_async_copy(emb_hbm.at[pl.ds(0, CHUNK)], rows.at[b],
                              sem_g[b]).wait()

    def issue_scatter(ci, b):
        base = w_base + ci * CHUNK
        pltpu.async_copy(rows.at[b], out_hbm.at[pl.ds(base, CHUNK)], sem_o[b])

    def wait_scatter(b):
        pltpu.make_async_copy(rows.at[b], out_hbm.at[pl.ds(0, CHUNK)],
                              sem_o[b]).wait()

    def add_pass(ci, b):
        @plsc.parallel_loop(0, CHUNK, 1, unroll=1)
        def add_row(r):
            c = idx_c[pl.ds(ci * CHUNK + r, LANES)][0]
            for k in range(DV):
                sl = pl.ds(k * LANES, LANES)
                plsc.addupdate(rows.at[b, r, sl], combo_v[c, sl])

    for j in range(LOOKAHEAD):
        issue_gather(j, j)

    def outer(go, carry):
        for b in range(NB):
            i = go * NB + b

            @pl.when(i + LOOKAHEAD < N_CHUNKS)
            def _next():
                nb = (b + LOOKAHEAD) % NB

                @pl.when(i + LOOKAHEAD >= NB)
                def _drain():
                    wait_scatter(nb)

                issue_gather(i + LOOKAHEAD, nb)

            wait_gather(b)
            add_pass(i, b)
            issue_scatter(i, b)
        return carry

    lax.fori_loop(0, N_CHUNKS // NB, outer, 0)
    for b in range(NB):
        wait_scatter(b)


@jax.jit
def _run(tok, pos, fpos, emb_table, pos_table, fpos_table):
    mesh = plsc.VectorSubcoreMesh(core_axis_name="c", subcore_axis_name="s")
    call = pl.kernel(
        _embed_kernel,
        mesh=mesh,
        out_type=jax.ShapeDtypeStruct((N, DIM), jnp.float32),
        scratch_types=[
            pltpu.VMEM((PER_W,), jnp.int32),
            pltpu.VMEM((PER_W + LANES,), jnp.int32),
            pltpu.VMEM((PER_W,), jnp.int32),
            pltpu.VMEM((NB, CHUNK, DIM), jnp.float32),
            pltpu.VMEM((N_POS, DIM), jnp.float32),
            pltpu.VMEM((N_FPOS, DIM), jnp.float32),
            pltpu.VMEM((N_COMBO, DIM), jnp.float32),
            pltpu.SemaphoreType.DMA,
            pltpu.SemaphoreType.DMA,
            pltpu.SemaphoreType.DMA,
            pltpu.SemaphoreType.DMA,
            pltpu.SemaphoreType.DMA,
            pltpu.SemaphoreType.DMA,
            pltpu.SemaphoreType.DMA,
            pltpu.SemaphoreType.DMA,
            pltpu.SemaphoreType.DMA,
        ],
    )
    out = call(tok, pos, fpos, emb_table, pos_table, fpos_table)
    return out.reshape(SEQ, BATCH, DIM)


def kernel(batch_datasets, batch_positionals, batch_float_positionals,
           emb_table, pos_table, fpos_table):
    tok = batch_datasets.reshape(N).astype(jnp.int32)
    pos = batch_positionals.reshape(N).astype(jnp.int32)
    fpos = batch_float_positionals.reshape(N).astype(jnp.int32)
    return _run(tok, pos, fpos, emb_table, pos_table, fpos_table)
